# baseline (speedup 1.0000x reference)
import functools

import jax
import jax.numpy as jnp
from jax import lax
from jax.experimental import pallas as pl
from jax.experimental.pallas import tpu as pltpu
from jax.experimental.pallas import tpu_sc as plsc

B = 16384
D = 32
L = 16
OUT_D = 3 * D


def kernel(gender_idx, age_idx, occupation_idx, W_gender, W_age, W_occupation):
    info = plsc.get_sparse_core_info()
    nw = info.num_cores * info.num_subcores
    b_per_w = B // nw
    n_groups = b_per_w // L
    mesh = plsc.VectorSubcoreMesh(core_axis_name="c", subcore_axis_name="s")

    @functools.partial(
        pl.kernel,
        mesh=mesh,
        out_type=jax.ShapeDtypeStruct((B, OUT_D), jnp.float32),
        compiler_params=pltpu.CompilerParams(needs_layout_passes=False,
                                             disable_bounds_checks=True,
                                             disable_semaphore_checks=True),
        scratch_types=[
            pltpu.VMEM((b_per_w,), jnp.int32),
            pltpu.VMEM((b_per_w,), jnp.int32),
            pltpu.VMEM((b_per_w,), jnp.int32),
            pltpu.VMEM((2, D), jnp.float32),
            pltpu.VMEM((7, D), jnp.float32),
            pltpu.VMEM((21, D), jnp.float32),
            pltpu.VMEM((b_per_w, OUT_D), jnp.float32),
            pltpu.SemaphoreType.DMA,
            pltpu.SemaphoreType.DMA,
        ],
    )
    def emb(g_hbm, a_hbm, o_hbm, wg_hbm, wa_hbm, wo_hbm, out_hbm,
            gi_v, ai_v, oi_v, tg_v, ta_v, to_v, stage_v, sem_in, sem_out):
        wid = lax.axis_index("s") * info.num_cores + lax.axis_index("c")
        base = wid * b_per_w
        copies = [
            pltpu.async_copy(g_hbm.at[pl.ds(base, b_per_w)], gi_v, sem_in),
            pltpu.async_copy(a_hbm.at[pl.ds(base, b_per_w)], ai_v, sem_in),
            pltpu.async_copy(o_hbm.at[pl.ds(base, b_per_w)], oi_v, sem_in),
            pltpu.async_copy(wg_hbm, tg_v, sem_in),
            pltpu.async_copy(wa_hbm, ta_v, sem_in),
            pltpu.async_copy(wo_hbm, to_v, sem_in),
        ]
        for c in copies:
            c.wait()

        n_chunks = 8
        gpc = n_groups // n_chunks
        rows_pc = gpc * L
        out_copies = []
        for chunk in range(n_chunks):

            @plsc.parallel_loop(chunk * gpc, (chunk + 1) * gpc, step=1,
                                unroll=2)
            def body(i):
                rows = (gi_v[pl.ds(i * L, L)],
                        ai_v[pl.ds(i * L, L)],
                        oi_v[pl.ds(i * L, L)])
                for l in range(L):
                    bidx = i * L + l
                    for f, t_v in enumerate((tg_v, ta_v, to_v)):
                        row = rows[f][l]
                        for h in range(2):
                            val = t_v[row, pl.ds(h * L, L)]
                            stage_v[bidx, pl.ds(f * D + h * L, L)] = val

            out_copies.append(pltpu.async_copy(
                stage_v.at[pl.ds(chunk * rows_pc, rows_pc)],
                out_hbm.at[pl.ds(base + chunk * rows_pc, rows_pc)],
                sem_out))
        for c in out_copies:
            c.wait()

    return emb(gender_idx, age_idx, occupation_idx,
               W_gender, W_age, W_occupation)


# speedup vs baseline: 1.1924x; 1.1924x over previous
import functools

import jax
import jax.numpy as jnp
from jax import lax
from jax.experimental import pallas as pl
from jax.experimental.pallas import tpu as pltpu
from jax.experimental.pallas import tpu_sc as plsc

B = 16384
D = 32
L = 16
OUT_D = 3 * D


def kernel(gender_idx, age_idx, occupation_idx, W_gender, W_age, W_occupation):
    info = plsc.get_sparse_core_info()
    nw = info.num_cores * info.num_subcores
    b_per_w = B // nw
    n_groups = b_per_w // L
    mesh = plsc.VectorSubcoreMesh(core_axis_name="c", subcore_axis_name="s")

    @functools.partial(
        pl.kernel,
        mesh=mesh,
        out_type=jax.ShapeDtypeStruct((B, OUT_D), jnp.float32),
        compiler_params=pltpu.CompilerParams(needs_layout_passes=False,
                                             disable_bounds_checks=True,
                                             disable_semaphore_checks=True),
        scratch_types=[
            pltpu.VMEM((b_per_w,), jnp.int32),
            pltpu.VMEM((b_per_w,), jnp.int32),
            pltpu.VMEM((b_per_w,), jnp.int32),
            pltpu.VMEM((2, D), jnp.float32),
            pltpu.VMEM((7, D), jnp.float32),
            pltpu.VMEM((21, D), jnp.float32),
            pltpu.VMEM((b_per_w, OUT_D), jnp.float32),
            pltpu.SemaphoreType.DMA,
            pltpu.SemaphoreType.DMA,
        ],
    )
    def emb(g_hbm, a_hbm, o_hbm, wg_hbm, wa_hbm, wo_hbm, out_hbm,
            gi_v, ai_v, oi_v, tg_v, ta_v, to_v, stage_v, sem_in, sem_out):
        wid = lax.axis_index("s") * info.num_cores + lax.axis_index("c")
        base = wid * b_per_w
        copies = [
            pltpu.async_copy(g_hbm.at[pl.ds(base, b_per_w)], gi_v, sem_in),
            pltpu.async_copy(a_hbm.at[pl.ds(base, b_per_w)], ai_v, sem_in),
            pltpu.async_copy(o_hbm.at[pl.ds(base, b_per_w)], oi_v, sem_in),
            pltpu.async_copy(wg_hbm, tg_v, sem_in),
            pltpu.async_copy(wa_hbm, ta_v, sem_in),
            pltpu.async_copy(wo_hbm, to_v, sem_in),
        ]
        for c in copies:
            c.wait()

        n_chunks = 2
        gpc = n_groups // n_chunks
        rows_pc = gpc * L
        out_copies = []
        for chunk in range(n_chunks):

            @plsc.parallel_loop(chunk * gpc, (chunk + 1) * gpc, step=1,
                                unroll=2)
            def body(i):
                rows = (gi_v[pl.ds(i * L, L)],
                        ai_v[pl.ds(i * L, L)],
                        oi_v[pl.ds(i * L, L)])
                for l in range(L):
                    bidx = i * L + l
                    for f, t_v in enumerate((tg_v, ta_v, to_v)):
                        row = rows[f][l]
                        for h in range(2):
                            val = t_v[row, pl.ds(h * L, L)]
                            stage_v[bidx, pl.ds(f * D + h * L, L)] = val

            out_copies.append(pltpu.async_copy(
                stage_v.at[pl.ds(chunk * rows_pc, rows_pc)],
                out_hbm.at[pl.ds(base + chunk * rows_pc, rows_pc)],
                sem_out))
        for c in out_copies:
            c.wait()

    return emb(gender_idx, age_idx, occupation_idx,
               W_gender, W_age, W_occupation)


# speedup vs baseline: 1.2914x; 1.0830x over previous
import functools

import jax
import jax.numpy as jnp
from jax import lax
from jax.experimental import pallas as pl
from jax.experimental.pallas import tpu as pltpu
from jax.experimental.pallas import tpu_sc as plsc

B = 16384
D = 32
L = 16
OUT_D = 3 * D


def kernel(gender_idx, age_idx, occupation_idx, W_gender, W_age, W_occupation):
    info = plsc.get_sparse_core_info()
    nw = info.num_cores * info.num_subcores
    b_per_w = B // nw
    n_groups = b_per_w // L
    mesh = plsc.VectorSubcoreMesh(core_axis_name="c", subcore_axis_name="s")

    @functools.partial(
        pl.kernel,
        mesh=mesh,
        out_type=jax.ShapeDtypeStruct((B, OUT_D), jnp.float32),
        compiler_params=pltpu.CompilerParams(needs_layout_passes=False,
                                             disable_bounds_checks=True,
                                             disable_semaphore_checks=True),
        scratch_types=[
            pltpu.VMEM((b_per_w,), jnp.int32),
            pltpu.VMEM((b_per_w,), jnp.int32),
            pltpu.VMEM((b_per_w,), jnp.int32),
            pltpu.VMEM((2, D), jnp.float32),
            pltpu.VMEM((7, D), jnp.float32),
            pltpu.VMEM((21, D), jnp.float32),
            pltpu.VMEM((b_per_w, OUT_D), jnp.float32),
            pltpu.SemaphoreType.DMA,
            pltpu.SemaphoreType.DMA,
        ],
    )
    def emb(g_hbm, a_hbm, o_hbm, wg_hbm, wa_hbm, wo_hbm, out_hbm,
            gi_v, ai_v, oi_v, tg_v, ta_v, to_v, stage_v, sem_in, sem_out):
        wid = lax.axis_index("s") * info.num_cores + lax.axis_index("c")
        base = wid * b_per_w
        copies = [
            pltpu.async_copy(g_hbm.at[pl.ds(base, b_per_w)], gi_v, sem_in),
            pltpu.async_copy(a_hbm.at[pl.ds(base, b_per_w)], ai_v, sem_in),
            pltpu.async_copy(o_hbm.at[pl.ds(base, b_per_w)], oi_v, sem_in),
            pltpu.async_copy(wg_hbm, tg_v, sem_in),
            pltpu.async_copy(wa_hbm, ta_v, sem_in),
            pltpu.async_copy(wo_hbm, to_v, sem_in),
        ]
        for c in copies:
            c.wait()

        n_chunks = 1
        gpc = n_groups // n_chunks
        rows_pc = gpc * L
        out_copies = []
        for chunk in range(n_chunks):

            @plsc.parallel_loop(chunk * gpc, (chunk + 1) * gpc, step=1,
                                unroll=2)
            def body(i):
                rows = (gi_v[pl.ds(i * L, L)],
                        ai_v[pl.ds(i * L, L)],
                        oi_v[pl.ds(i * L, L)])
                for l in range(L):
                    bidx = i * L + l
                    for f, t_v in enumerate((tg_v, ta_v, to_v)):
                        row = rows[f][l]
                        for h in range(2):
                            val = t_v[row, pl.ds(h * L, L)]
                            stage_v[bidx, pl.ds(f * D + h * L, L)] = val

            out_copies.append(pltpu.async_copy(
                stage_v.at[pl.ds(chunk * rows_pc, rows_pc)],
                out_hbm.at[pl.ds(base + chunk * rows_pc, rows_pc)],
                sem_out))
        for c in out_copies:
            c.wait()

    return emb(gender_idx, age_idx, occupation_idx,
               W_gender, W_age, W_occupation)


# speedup vs baseline: 1.3331x; 1.0323x over previous
import functools

import jax
import jax.numpy as jnp
from jax import lax
from jax.experimental import pallas as pl
from jax.experimental.pallas import tpu as pltpu
from jax.experimental.pallas import tpu_sc as plsc

B = 16384
D = 32
L = 16
OUT_D = 3 * D


def kernel(gender_idx, age_idx, occupation_idx, W_gender, W_age, W_occupation):
    info = plsc.get_sparse_core_info()
    nw = info.num_cores * info.num_subcores
    b_per_w = B // nw
    n_groups = b_per_w // L
    mesh = plsc.VectorSubcoreMesh(core_axis_name="c", subcore_axis_name="s")

    @functools.partial(
        pl.kernel,
        mesh=mesh,
        out_type=jax.ShapeDtypeStruct((B, OUT_D), jnp.float32),
        compiler_params=pltpu.CompilerParams(needs_layout_passes=False,
                                             disable_bounds_checks=True,
                                             disable_semaphore_checks=True),
        scratch_types=[
            pltpu.VMEM((b_per_w,), jnp.int32),
            pltpu.VMEM((b_per_w,), jnp.int32),
            pltpu.VMEM((b_per_w,), jnp.int32),
            pltpu.VMEM((2, D), jnp.float32),
            pltpu.VMEM((7, D), jnp.float32),
            pltpu.VMEM((21, D), jnp.float32),
            pltpu.VMEM((b_per_w, OUT_D), jnp.float32),
            pltpu.SemaphoreType.DMA,
            pltpu.SemaphoreType.DMA,
        ],
    )
    def emb(g_hbm, a_hbm, o_hbm, wg_hbm, wa_hbm, wo_hbm, out_hbm,
            gi_v, ai_v, oi_v, tg_v, ta_v, to_v, stage_v, sem_in, sem_out):
        wid = lax.axis_index("s") * info.num_cores + lax.axis_index("c")
        base = wid * b_per_w
        copies = [
            pltpu.async_copy(g_hbm.at[pl.ds(base, b_per_w)], gi_v, sem_in),
            pltpu.async_copy(a_hbm.at[pl.ds(base, b_per_w)], ai_v, sem_in),
            pltpu.async_copy(o_hbm.at[pl.ds(base, b_per_w)], oi_v, sem_in),
            pltpu.async_copy(wg_hbm, tg_v, sem_in),
            pltpu.async_copy(wa_hbm, ta_v, sem_in),
            pltpu.async_copy(wo_hbm, to_v, sem_in),
        ]
        for c in copies:
            c.wait()

        n_chunks = 1
        gpc = n_groups // n_chunks
        rows_pc = gpc * L
        out_copies = []
        for chunk in range(n_chunks):

            @plsc.parallel_loop(chunk * gpc, (chunk + 1) * gpc, step=1,
                                unroll=1)
            def body(i):
                rows = (gi_v[pl.ds(i * L, L)],
                        ai_v[pl.ds(i * L, L)],
                        oi_v[pl.ds(i * L, L)])
                for l in range(L):
                    bidx = i * L + l
                    for f, t_v in enumerate((tg_v, ta_v, to_v)):
                        row = rows[f][l]
                        for h in range(2):
                            val = t_v[row, pl.ds(h * L, L)]
                            stage_v[bidx, pl.ds(f * D + h * L, L)] = val

            out_copies.append(pltpu.async_copy(
                stage_v.at[pl.ds(chunk * rows_pc, rows_pc)],
                out_hbm.at[pl.ds(base + chunk * rows_pc, rows_pc)],
                sem_out))
        for c in out_copies:
            c.wait()

    return emb(gender_idx, age_idx, occupation_idx,
               W_gender, W_age, W_occupation)
